# Initial kernel scaffold; baseline (speedup 1.0000x reference)
import jax
import jax.numpy as jnp
from jax.experimental import pallas as pl


def kernel(input, embeddings):
    raise NotImplementedError("write your pallas kernel here")


# speedup vs baseline: 1.0234x; 1.0234x over previous
import functools

import jax
import jax.numpy as jnp
from jax import lax
from jax.experimental import pallas as pl
from jax.experimental.pallas import tpu as pltpu
from jax.experimental.pallas import tpu_sc as plsc

D = 32

_info = plsc.get_sparse_core_info()
_NC, _NS = _info.num_cores, _info.num_subcores
_NW = _NC * _NS

_CHUNK = 128


def _make_gather(total_rows: int):
    per_w = total_rows // _NW
    nchunk = per_w // _CHUNK
    mesh = plsc.VectorSubcoreMesh(core_axis_name="c", subcore_axis_name="s")

    @functools.partial(
        pl.kernel,
        out_type=jax.ShapeDtypeStruct((total_rows, D), jnp.float32),
        mesh=mesh,
        scratch_types=[
            pltpu.VMEM((per_w,), jnp.int32),
            pltpu.VMEM((_CHUNK, D), jnp.float32),
            pltpu.SemaphoreType.DMA,
        ],
        compiler_params=pltpu.CompilerParams(use_tc_tiling_on_sc=False),
    )
    def gather_kernel(idx_hbm, table_hbm, out_hbm, idx_v, rows_v, sem):
        wid = lax.axis_index("s") * _NC + lax.axis_index("c")
        base = wid * per_w
        pltpu.sync_copy(idx_hbm.at[pl.ds(base, per_w)], idx_v)

        def body(c, carry):
            off = c * _CHUNK
            pltpu.async_copy(
                table_hbm.at[idx_v.at[pl.ds(off, _CHUNK)]], rows_v, sem
            ).wait()
            pltpu.sync_copy(rows_v, out_hbm.at[pl.ds(base + off, _CHUNK)])
            return carry

        lax.fori_loop(0, nchunk, body, 0)

    return gather_kernel


def kernel(input, embeddings):
    idx = input.reshape(-1).astype(jnp.int32)
    out = _make_gather(idx.shape[0])(idx, embeddings)
    return out.reshape(input.shape + (D,))


# speedup vs baseline: 1.1123x; 1.0868x over previous
import functools

import jax
import jax.numpy as jnp
from jax import lax
from jax.experimental import pallas as pl
from jax.experimental.pallas import tpu as pltpu
from jax.experimental.pallas import tpu_sc as plsc

D = 32

_info = plsc.get_sparse_core_info()
_NC, _NS = _info.num_cores, _info.num_subcores
_NW = _NC * _NS

_CHUNK = 128
_NBUF = 8


def _make_gather(total_rows: int):
    per_w = total_rows // _NW
    nchunk = per_w // _CHUNK
    ngroup = nchunk // _NBUF
    assert ngroup * _NBUF == nchunk
    mesh = plsc.VectorSubcoreMesh(core_axis_name="c", subcore_axis_name="s")

    @functools.partial(
        pl.kernel,
        out_type=jax.ShapeDtypeStruct((total_rows, D), jnp.float32),
        mesh=mesh,
        scratch_types=[
            pltpu.VMEM((per_w,), jnp.int32),
            pltpu.VMEM((_NBUF, _CHUNK, D), jnp.float32),
            pltpu.SemaphoreType.DMA((_NBUF,)),
            pltpu.SemaphoreType.DMA((_NBUF,)),
        ],
        compiler_params=pltpu.CompilerParams(use_tc_tiling_on_sc=False),
    )
    def gather_kernel(idx_hbm, table_hbm, out_hbm, idx_v, rows_v, in_sem, out_sem):
        wid = lax.axis_index("s") * _NC + lax.axis_index("c")
        base = wid * per_w
        pltpu.sync_copy(idx_hbm.at[pl.ds(base, per_w)], idx_v)

        def gather_chunk(c, b):
            pltpu.async_copy(
                table_hbm.at[idx_v.at[pl.ds(c * _CHUNK, _CHUNK)]],
                rows_v.at[b],
                in_sem.at[b],
            )

        def body(g, carry):
            for b in range(_NBUF):
                c = g * _NBUF + b
                pltpu.make_async_copy(
                    table_hbm.at[idx_v.at[pl.ds(c * _CHUNK, _CHUNK)]],
                    rows_v.at[b],
                    in_sem.at[b],
                ).wait()
                pltpu.async_copy(
                    rows_v.at[b],
                    out_hbm.at[pl.ds(base + c * _CHUNK, _CHUNK)],
                    out_sem.at[b],
                )
            @pl.when(g + 1 < ngroup)
            def _():
                for b in range(_NBUF):
                    c = (g + 1) * _NBUF + b
                    pltpu.make_async_copy(
                        rows_v.at[b],
                        out_hbm.at[pl.ds(base, _CHUNK)],
                        out_sem.at[b],
                    ).wait()
                    gather_chunk(c, b)

            return carry

        for b in range(_NBUF):
            gather_chunk(b, b)
        lax.fori_loop(0, ngroup, body, 0)
        for b in range(_NBUF):
            pltpu.make_async_copy(
                rows_v.at[b],
                out_hbm.at[pl.ds(base, _CHUNK)],
                out_sem.at[b],
            ).wait()

    return gather_kernel


def kernel(input, embeddings):
    idx = input.reshape(-1).astype(jnp.int32)
    out = _make_gather(idx.shape[0])(idx, embeddings)
    return out.reshape(input.shape + (D,))


# speedup vs baseline: 1.8007x; 1.6190x over previous
import functools

import jax
import jax.numpy as jnp
from jax import lax
from jax.experimental import pallas as pl
from jax.experimental.pallas import tpu as pltpu
from jax.experimental.pallas import tpu_sc as plsc

D = 32

_info = plsc.get_sparse_core_info()
_NC, _NS = _info.num_cores, _info.num_subcores
_NW = _NC * _NS

_G = 8
_NRING = 4


def _make_gather(n_samples: int, seq: int):
    per_w = n_samples // _NW
    ngroup = per_w // _G
    nsuper = ngroup // _NRING
    assert nsuper * _NRING * _G == per_w
    mesh = plsc.VectorSubcoreMesh(core_axis_name="c", subcore_axis_name="s")

    @functools.partial(
        pl.kernel,
        out_type=jax.ShapeDtypeStruct((n_samples, seq, D), jnp.float32),
        mesh=mesh,
        scratch_types=[
            pltpu.VMEM((per_w, seq), jnp.int32),
            [pltpu.VMEM((_G, seq, D), jnp.float32) for _ in range(_NRING)],
            pltpu.SemaphoreType.DMA((_NRING,)),
            pltpu.SemaphoreType.DMA((_NRING,)),
        ],
        compiler_params=pltpu.CompilerParams(use_tc_tiling_on_sc=False),
    )
    def gather_kernel(idx_hbm, table_hbm, out_hbm, idx_v, rows, in_sem, out_sem):
        wid = lax.axis_index("s") * _NC + lax.axis_index("c")
        r0 = wid * per_w
        pltpu.sync_copy(idx_hbm.at[pl.ds(r0, per_w)], idx_v)

        def fire_group(g, b):
            for k in range(_G):
                pltpu.async_copy(
                    table_hbm.at[idx_v.at[g * _G + k]],
                    rows[b].at[k],
                    in_sem.at[b],
                )

        def drain_group(b):
            for k in range(_G):
                pltpu.make_async_copy(
                    table_hbm.at[idx_v.at[k]],
                    rows[b].at[k],
                    in_sem.at[b],
                ).wait()

        def wait_out(b):
            pltpu.make_async_copy(
                rows[b],
                out_hbm.at[pl.ds(r0, _G)],
                out_sem.at[b],
            ).wait()

        def body(sg, carry):
            for b in range(_NRING):
                g = sg * _NRING + b
                drain_group(b)
                pltpu.async_copy(
                    rows[b],
                    out_hbm.at[pl.ds(r0 + g * _G, _G)],
                    out_sem.at[b],
                )

            @pl.when(sg + 1 < nsuper)
            def _():
                for b in range(_NRING):
                    g = (sg + 1) * _NRING + b
                    wait_out(b)
                    fire_group(g, b)

            return carry

        for b in range(_NRING):
            fire_group(b, b)
        lax.fori_loop(0, nsuper, body, 0)
        for b in range(_NRING):
            wait_out(b)

    return gather_kernel


def kernel(input, embeddings):
    n_samples, seq = input.shape
    return _make_gather(n_samples, seq)(input, embeddings)


# speedup vs baseline: 2.5363x; 1.4084x over previous
import functools

import jax
import jax.numpy as jnp
from jax import lax
from jax.experimental import pallas as pl
from jax.experimental.pallas import tpu as pltpu
from jax.experimental.pallas import tpu_sc as plsc

D = 32

_info = plsc.get_sparse_core_info()
_NC, _NS = _info.num_cores, _info.num_subcores
_NW = _NC * _NS

_G = 8
_NRING = 4


def _make_gather(n_samples: int, seq: int):
    per_w = n_samples // _NW
    ngroup = per_w // _G
    nsuper = ngroup // _NRING
    assert nsuper * _NRING * _G == per_w
    mesh = plsc.VectorSubcoreMesh(core_axis_name="c", subcore_axis_name="s")

    seq_p, d_p = 56, 128

    @functools.partial(
        pl.kernel,
        out_type=jax.ShapeDtypeStruct((n_samples, seq_p, d_p), jnp.float32),
        mesh=mesh,
        scratch_types=[
            pltpu.VMEM((per_w, seq), jnp.int32),
            [pltpu.VMEM((_G, seq, D), jnp.float32) for _ in range(_NRING)],
            pltpu.SemaphoreType.DMA((_NRING,)),
            pltpu.SemaphoreType.DMA((_NRING,)),
        ],
        compiler_params=pltpu.CompilerParams(use_tc_tiling_on_sc=False),
    )
    def gather_kernel(idx_hbm, table_hbm, out_hbm, idx_v, rows, in_sem, out_sem):
        wid = lax.axis_index("s") * _NC + lax.axis_index("c")
        r0 = wid * per_w
        pltpu.sync_copy(idx_hbm.at[pl.ds(r0, per_w)], idx_v)

        def fire_group(g, b):
            for k in range(_G):
                pltpu.async_copy(
                    table_hbm.at[idx_v.at[g * _G + k]],
                    rows[b].at[k],
                    in_sem.at[b],
                )

        def drain_group(b):
            for k in range(_G):
                pltpu.make_async_copy(
                    table_hbm.at[idx_v.at[k]],
                    rows[b].at[k],
                    in_sem.at[b],
                ).wait()

        def out_slice(g):
            return out_hbm.at[pl.ds(r0 + g * _G, _G), pl.ds(0, seq), pl.ds(0, D)]

        def wait_out(b):
            pltpu.make_async_copy(rows[b], out_slice(0), out_sem.at[b]).wait()

        def body(sg, carry):
            for b in range(_NRING):
                g = sg * _NRING + b
                drain_group(b)
                pltpu.async_copy(rows[b], out_slice(g), out_sem.at[b])

            @pl.when(sg + 1 < nsuper)
            def _():
                for b in range(_NRING):
                    g = (sg + 1) * _NRING + b
                    wait_out(b)
                    fire_group(g, b)

            return carry

        for b in range(_NRING):
            fire_group(b, b)
        lax.fori_loop(0, nsuper, body, 0)
        for b in range(_NRING):
            wait_out(b)

    return gather_kernel


def kernel(input, embeddings):
    n_samples, seq = input.shape
    out_padded = _make_gather(n_samples, seq)(input, embeddings)
    return out_padded[:, :seq, :D]
